# baseline (speedup 1.0000x reference)
import functools

import jax
import jax.numpy as jnp
from jax import lax
from jax.experimental import pallas as pl
from jax.experimental.pallas import tpu as pltpu
from jax.experimental.pallas import tpu_sc as plsc

_VOCAB = 1000000
_EMB = 32
_BATCH = 16384
_HIST = 50
_B = _BATCH * _HIST
_NC = 2
_NS = 16
_NW = _NC * _NS
_BPW = _B // _NW
_CHUNK = 320
_NCHUNK = _BPW // _CHUNK
_NB = 2
_NSTEP = _NCHUNK // _NB
_G = _CHUNK // 16


@functools.partial(
    pl.kernel,
    mesh=plsc.VectorSubcoreMesh(core_axis_name="c", subcore_axis_name="s"),
    out_type=jax.ShapeDtypeStruct((_B * _EMB,), jnp.float32),
    scratch_types=(
        [pltpu.VMEM((_CHUNK,), jnp.int32) for _ in range(_NB)]
        + [pltpu.VMEM((_CHUNK,), jnp.int32) for _ in range(_NB)]
        + [pltpu.VMEM((_CHUNK, 4 * _EMB), jnp.float32) for _ in range(_NB)]
        + [pltpu.VMEM((_CHUNK * _EMB,), jnp.float32) for _ in range(_NB)]
        + [pltpu.SemaphoreType.DMA for _ in range(2 * _NB)]
    ),
    compiler_params=pltpu.CompilerParams(needs_layout_passes=False),
)
def _sc_gather(idx_hbm, table_hbm, out_hbm, *scratch):
    idx_v = scratch[:_NB]
    bid_v = scratch[_NB:2 * _NB]
    blk_v = scratch[2 * _NB:3 * _NB]
    stage_v = scratch[3 * _NB:4 * _NB]
    g_sem = scratch[4 * _NB:5 * _NB]
    st_sem = scratch[5 * _NB:6 * _NB]
    wid = lax.axis_index("s") * _NC + lax.axis_index("c")
    base = wid * _BPW

    def prep_chunk(i, b):
        pltpu.sync_copy(idx_hbm.at[pl.ds(base + i * _CHUNK, _CHUNK)],
                        idx_v[b])

        def bids(g, carry):
            iv = idx_v[b][pl.ds(g * 16, 16)]
            bid_v[b][pl.ds(g * 16, 16)] = lax.shift_right_logical(iv, 2)
            return carry

        lax.fori_loop(0, _G, bids, 0)
        pltpu.async_copy(table_hbm.at[bid_v[b]], blk_v[b], g_sem[b])

    def wait_gather(b):
        pltpu.make_async_copy(table_hbm.at[pl.ds(0, _CHUNK)], blk_v[b],
                              g_sem[b]).wait()

    def extract(b):
        def grp(g, carry):
            rows = g * 16 + lax.iota(jnp.int32, 16)
            iv = idx_v[b][pl.ds(g * 16, 16)]
            sub = (iv & 3) * 32
            rows_t = rows * _EMB
            for c in range(_EMB):
                vals = plsc.load_gather(blk_v[b], [rows, sub + c])
                plsc.store_scatter(stage_v[b], [rows_t + c], vals)
            return carry

        lax.fori_loop(0, _G, grp, 0)

    def start_store(i, b):
        pltpu.async_copy(stage_v[b],
                         out_hbm.at[pl.ds((base + i * _CHUNK) * _EMB,
                                          _CHUNK * _EMB)],
                         st_sem[b])

    def wait_store(b):
        pltpu.make_async_copy(stage_v[b],
                              out_hbm.at[pl.ds(0, _CHUNK * _EMB)],
                              st_sem[b]).wait()

    for b in range(_NB):
        prep_chunk(b, b)

    def body(j, carry):
        for b in range(_NB):
            i = j * _NB + b
            wait_gather(b)

            @pl.when(j > 0)
            def _():
                wait_store(b)

            extract(b)
            start_store(i, b)
            prep_chunk(i + _NB, b)
        return carry

    lax.fori_loop(0, _NSTEP - 1, body, 0)

    for b in range(_NB):
        i = (_NSTEP - 1) * _NB + b
        wait_gather(b)
        wait_store(b)
        extract(b)
        start_store(i, b)
    for b in range(_NB):
        wait_store(b)


def kernel(inputs, V):
    flat_idx = inputs.reshape(_B)
    table = V.reshape(_VOCAB // 4, 4 * _EMB)
    out = _sc_gather(flat_idx, table)
    return out.reshape(_BATCH, _HIST, _EMB)


# speedup vs baseline: 1.1706x; 1.1706x over previous
import functools

import jax
import jax.numpy as jnp
from jax import lax
from jax.experimental import pallas as pl
from jax.experimental.pallas import tpu as pltpu
from jax.experimental.pallas import tpu_sc as plsc

_VOCAB = 1000000
_EMB = 32
_BATCH = 16384
_HIST = 50
_B = _BATCH * _HIST
_NC = 2
_NS = 16
_NW = _NC * _NS
_BPW = _B // _NW
_CHUNK = 800
_NCHUNK = _BPW // _CHUNK
_NB = 4
_NSTEP = _NCHUNK // _NB


@functools.partial(
    pl.kernel,
    mesh=plsc.VectorSubcoreMesh(core_axis_name="c", subcore_axis_name="s"),
    out_type=jax.ShapeDtypeStruct((_B, _EMB), jnp.float32),
    scratch_types=(
        [pltpu.VMEM((_CHUNK,), jnp.int32) for _ in range(_NB)]
        + [pltpu.VMEM((_CHUNK, _EMB), jnp.float32) for _ in range(_NB)]
        + [pltpu.SemaphoreType.DMA for _ in range(2 * _NB)]
    ),
    compiler_params=pltpu.CompilerParams(use_tc_tiling_on_sc=False),
)
def _sc_gather(idx_hbm, table_hbm, out_hbm, *scratch):
    idx_v = scratch[:_NB]
    rows_v = scratch[_NB:2 * _NB]
    g_sem = scratch[2 * _NB:3 * _NB]
    st_sem = scratch[3 * _NB:4 * _NB]
    wid = lax.axis_index("s") * _NC + lax.axis_index("c")
    base = wid * _BPW

    def load_idx(i, b):
        pltpu.sync_copy(idx_hbm.at[pl.ds(base + i * _CHUNK, _CHUNK)],
                        idx_v[b])

    def start_gather(b):
        pltpu.async_copy(table_hbm.at[idx_v[b]], rows_v[b], g_sem[b])

    def wait_gather(b):
        pltpu.make_async_copy(table_hbm.at[pl.ds(0, _CHUNK)], rows_v[b],
                              g_sem[b]).wait()

    def start_store(i, b):
        pltpu.async_copy(rows_v[b],
                         out_hbm.at[pl.ds(base + i * _CHUNK, _CHUNK)],
                         st_sem[b])

    def wait_store(b):
        pltpu.make_async_copy(rows_v[b], out_hbm.at[pl.ds(0, _CHUNK)],
                              st_sem[b]).wait()

    for b in range(_NB):
        load_idx(b, b)
        start_gather(b)

    def body(j, carry):
        for b in range(_NB):
            i = j * _NB + b
            wait_gather(b)
            start_store(i, b)
            load_idx(i + _NB, b)
            wait_store(b)
            start_gather(b)
        return carry

    lax.fori_loop(0, _NSTEP - 1, body, 0)

    for b in range(_NB):
        i = (_NSTEP - 1) * _NB + b
        wait_gather(b)
        start_store(i, b)
    for b in range(_NB):
        wait_store(b)


def kernel(inputs, V):
    flat_idx = inputs.reshape(_B)
    out = _sc_gather(flat_idx, V)
    return out.reshape(_BATCH, _HIST, _EMB)
